# baseline (speedup 1.0000x reference)
import functools

import jax
import jax.numpy as jnp
from jax import lax
from jax.experimental import pallas as pl
from jax.experimental.pallas import tpu as pltpu
from jax.experimental.pallas import tpu_sc as plsc

N = 10000
D = 128
E = 320000

NC = 2
NS = 16
NW = NC * NS

G = 128
K = 4

Q = 80
NGROUPS = E // G
QL = NGROUPS - (NW - 1) * Q

ACC_ROWS = 10240
R_PT = ACC_ROWS // NS


def _sc_body(ei_hbm, nid_hbm, emb_hbm, sums_hbm, cnts_hbm,
             src_v, dst_v, map_v, rows_v, ones_v, z16_v, zc_v, acc_s,
             cnt_s, sem_i, sem_m, sem_r):
    cid = lax.axis_index("c")
    sid = lax.axis_index("s")
    r0 = sid * R_PT

    w = cid * NS + sid
    gbase = w * Q
    last = w == NW - 1
    nsg = jnp.where(last, QL // K, Q // K)
    npairs = jnp.where(last, QL // K // 2, Q // K // 2)

    for j in range(G // 16):
        ones_v[pl.ds(j * 16, 16)] = jnp.ones((16,), jnp.float32)
    for i in range(16):
        for j in range(D // 16):
            z16_v[i, pl.ds(j * 16, 16)] = jnp.zeros((16,), jnp.float32)

    with jax.named_scope("ph_zero"):
        def zc_row(k, carry):
            o = pl.multiple_of(k * 16, 16)
            zc_v[pl.ds(o, 16)] = jnp.zeros((16,), jnp.float32)
            return carry

        lax.fori_loop(0, R_PT // 16, zc_row, 0)

        def zrow(k, carry):
            pltpu.sync_copy(z16_v, acc_s.at[pl.ds(r0 + k * 16, 16), :])
            return carry

        lax.fori_loop(0, R_PT // 16, zrow, 0)
        pltpu.sync_copy(zc_v, cnt_s.at[pl.ds(r0, R_PT)])

    with jax.named_scope("ph_bar0"):
        plsc.subcore_barrier()

    def do_sg(i, p):
        gn = jnp.minimum(gbase + (i + 1) * K, gbase + (nsg - 1) * K)
        cs = pltpu.async_copy(ei_hbm.at[0, pl.ds(gn, K)], src_v.at[1 - p], sem_i)
        cd = pltpu.async_copy(ei_hbm.at[1, pl.ds(gn, K)], dst_v.at[1 - p], sem_i)
        cs.wait()
        cd.wait()
        mcs = [
            pltpu.async_copy(
                nid_hbm.at[src_v.at[1 - p, k]], map_v.at[1 - p, k],
                sem_m.at[1 - p])
            for k in range(K)
        ]
        rcs = [None] * K
        for b in range(2):
            rcs[b] = pltpu.async_copy(
                emb_hbm.at[map_v.at[p, b]], rows_v.at[b], sem_r.at[b])
        for k in range(K):
            rcs[k].wait()
            pltpu.sync_copy(rows_v.at[k % 2], acc_s.at[dst_v.at[p, k]], add=True)
            pltpu.sync_copy(ones_v, cnt_s.at[dst_v.at[p, k]], add=True)
            if k + 2 < K:
                rcs[k + 2] = pltpu.async_copy(
                    emb_hbm.at[map_v.at[p, k + 2]], rows_v.at[k % 2],
                    sem_r.at[k % 2])
        for cp in mcs:
            cp.wait()

    def super_pair(j, carry):
        do_sg(2 * j, 0)
        do_sg(2 * j + 1, 1)
        return carry

    with jax.named_scope("ph_main"):
        pltpu.sync_copy(ei_hbm.at[0, pl.ds(gbase, K)], src_v.at[0])
        pltpu.sync_copy(ei_hbm.at[1, pl.ds(gbase, K)], dst_v.at[0])
        mcs0 = [
            pltpu.async_copy(
                nid_hbm.at[src_v.at[0, k]], map_v.at[0, k], sem_m.at[0])
            for k in range(K)
        ]
        for cp in mcs0:
            cp.wait()
        lax.fori_loop(0, npairs, super_pair, 0)
        pl.when(last)(lambda: do_sg(QL // K - 1, 0))

    with jax.named_scope("ph_bar1"):
        plsc.subcore_barrier()

    with jax.named_scope("ph_wb"):
        pltpu.sync_copy(acc_s.at[pl.ds(r0, R_PT), :], sums_hbm.at[cid, pl.ds(r0, R_PT), :])
        pltpu.sync_copy(cnt_s.at[pl.ds(r0, R_PT)], cnts_hbm.at[cid, pl.ds(r0, R_PT)])


_sc_accumulate = functools.partial(
    pl.kernel,
    mesh=plsc.VectorSubcoreMesh(core_axis_name="c", subcore_axis_name="s"),
    out_type=[
        jax.ShapeDtypeStruct((NC, ACC_ROWS, D), jnp.float32),
        jax.ShapeDtypeStruct((NC, ACC_ROWS), jnp.float32),
    ],
    scratch_types=[
        pltpu.VMEM((2, K, G), jnp.int32),
        pltpu.VMEM((2, K, G), jnp.int32),
        pltpu.VMEM((2, K, G), jnp.int32),
        pltpu.VMEM((2, G, D), jnp.float32),
        pltpu.VMEM((G,), jnp.float32),
        pltpu.VMEM((16, D), jnp.float32),
        pltpu.VMEM((R_PT,), jnp.float32),
        pltpu.VMEM_SHARED((ACC_ROWS, D), jnp.float32),
        pltpu.VMEM_SHARED((ACC_ROWS,), jnp.float32),
        pltpu.SemaphoreType.DMA,
        pltpu.SemaphoreType.DMA((2,)),
        pltpu.SemaphoreType.DMA((2,)),
    ],
)(_sc_body)


BLK = 2000


def _fin_body(s_ref, c_ref, o_ref):
    s = s_ref[0] + s_ref[1]
    c = c_ref[0] + c_ref[1]
    o_ref[...] = s / jnp.maximum(c, 1.0)


_finalize = pl.pallas_call(
    _fin_body,
    grid=(N // BLK,),
    in_specs=[
        pl.BlockSpec((NC, BLK, D), lambda i: (0, i, 0)),
        pl.BlockSpec((NC, BLK, 1), lambda i: (0, i, 0)),
    ],
    out_specs=pl.BlockSpec((BLK, D), lambda i: (i, 0)),
    out_shape=jax.ShapeDtypeStruct((N, D), jnp.float32),
)


def kernel(node_ids, edge_index, emb_weight):
    ei_p = edge_index.reshape(2, NGROUPS, G)
    sums, cnts = _sc_accumulate(ei_p, node_ids, emb_weight)
    return _finalize(sums, cnts.reshape(NC, ACC_ROWS, 1))


# speedup vs baseline: 1.0722x; 1.0722x over previous
import functools

import jax
import jax.numpy as jnp
from jax import lax
from jax.experimental import pallas as pl
from jax.experimental.pallas import tpu as pltpu
from jax.experimental.pallas import tpu_sc as plsc

N = 10000
D = 128
E = 320000

NC = 2
NS = 16
NW = NC * NS

G = 128
K = 8
KL = 4

Q = 80
NGROUPS = E // G
QL = NGROUPS - (NW - 1) * Q

ACC_ROWS = 10240
R_PT = ACC_ROWS // NS


def _sc_body(ei_hbm, nid_hbm, emb_hbm, sums_hbm, cnts_hbm,
             src_v, dst_v, map_v, rows_v, ones_v, z16_v, zc_v, acc_s,
             cnt_s, sem_i, sem_m, sem_r):
    cid = lax.axis_index("c")
    sid = lax.axis_index("s")
    r0 = sid * R_PT

    w = cid * NS + sid
    gbase = w * Q
    last = w == NW - 1

    for j in range(G // 16):
        ones_v[pl.ds(j * 16, 16)] = jnp.ones((16,), jnp.float32)
    for i in range(16):
        for j in range(D // 16):
            z16_v[i, pl.ds(j * 16, 16)] = jnp.zeros((16,), jnp.float32)

    with jax.named_scope("ph_zero"):
        def zc_row(k, carry):
            o = pl.multiple_of(k * 16, 16)
            zc_v[pl.ds(o, 16)] = jnp.zeros((16,), jnp.float32)
            return carry

        lax.fori_loop(0, R_PT // 16, zc_row, 0)

        def zrow(k, carry):
            pltpu.sync_copy(z16_v, acc_s.at[pl.ds(r0 + k * 16, 16), :])
            return carry

        lax.fori_loop(0, R_PT // 16, zrow, 0)
        pltpu.sync_copy(zc_v, cnt_s.at[pl.ds(r0, R_PT)])

    with jax.named_scope("ph_bar0"):
        plsc.subcore_barrier()

    def make_do_sg(KK, nsg_s):
        def do_sg(i, p):
            gn = jnp.minimum(gbase + (i + 1) * KK, gbase + (nsg_s - 1) * KK)
            cs = pltpu.async_copy(
                ei_hbm.at[0, pl.ds(gn, KK)], src_v.at[1 - p, pl.ds(0, KK)],
                sem_i)
            cd = pltpu.async_copy(
                ei_hbm.at[1, pl.ds(gn, KK)], dst_v.at[1 - p, pl.ds(0, KK)],
                sem_i)
            cs.wait()
            cd.wait()
            mcs = [
                pltpu.async_copy(
                    nid_hbm.at[src_v.at[1 - p, k]], map_v.at[1 - p, k],
                    sem_m.at[1 - p])
                for k in range(KK)
            ]
            rcs = [None] * KK
            for b in range(2):
                rcs[b] = pltpu.async_copy(
                    emb_hbm.at[map_v.at[p, b]], rows_v.at[b], sem_r.at[b])
            for k in range(KK):
                rcs[k].wait()
                pltpu.sync_copy(rows_v.at[k % 2], acc_s.at[dst_v.at[p, k]],
                                add=True)
                pltpu.sync_copy(ones_v, cnt_s.at[dst_v.at[p, k]], add=True)
                if k + 2 < KK:
                    rcs[k + 2] = pltpu.async_copy(
                        emb_hbm.at[map_v.at[p, k + 2]], rows_v.at[k % 2],
                        sem_r.at[k % 2])
            for cp in mcs:
                cp.wait()
        return do_sg

    def run_path(KK, nsg_s):
        do_sg = make_do_sg(KK, nsg_s)
        pltpu.sync_copy(ei_hbm.at[0, pl.ds(gbase, KK)],
                        src_v.at[0, pl.ds(0, KK)])
        pltpu.sync_copy(ei_hbm.at[1, pl.ds(gbase, KK)],
                        dst_v.at[0, pl.ds(0, KK)])
        mcs0 = [
            pltpu.async_copy(
                nid_hbm.at[src_v.at[0, k]], map_v.at[0, k], sem_m.at[0])
            for k in range(KK)
        ]
        for cp in mcs0:
            cp.wait()

        def super_pair(j, carry):
            do_sg(2 * j, 0)
            do_sg(2 * j + 1, 1)
            return carry

        lax.fori_loop(0, nsg_s // 2, super_pair, 0)
        if nsg_s % 2:
            do_sg(nsg_s - 1, 0)

    with jax.named_scope("ph_main"):
        pl.when(jnp.logical_not(last))(lambda: run_path(K, Q // K))
        pl.when(last)(lambda: run_path(KL, QL // KL))

    with jax.named_scope("ph_bar1"):
        plsc.subcore_barrier()

    with jax.named_scope("ph_wb"):
        pltpu.sync_copy(acc_s.at[pl.ds(r0, R_PT), :], sums_hbm.at[cid, pl.ds(r0, R_PT), :])
        pltpu.sync_copy(cnt_s.at[pl.ds(r0, R_PT)], cnts_hbm.at[cid, pl.ds(r0, R_PT)])


_sc_accumulate = functools.partial(
    pl.kernel,
    mesh=plsc.VectorSubcoreMesh(core_axis_name="c", subcore_axis_name="s"),
    out_type=[
        jax.ShapeDtypeStruct((NC, ACC_ROWS, D), jnp.float32),
        jax.ShapeDtypeStruct((NC, ACC_ROWS), jnp.float32),
    ],
    scratch_types=[
        pltpu.VMEM((2, K, G), jnp.int32),
        pltpu.VMEM((2, K, G), jnp.int32),
        pltpu.VMEM((2, K, G), jnp.int32),
        pltpu.VMEM((2, G, D), jnp.float32),
        pltpu.VMEM((G,), jnp.float32),
        pltpu.VMEM((16, D), jnp.float32),
        pltpu.VMEM((R_PT,), jnp.float32),
        pltpu.VMEM_SHARED((ACC_ROWS, D), jnp.float32),
        pltpu.VMEM_SHARED((ACC_ROWS,), jnp.float32),
        pltpu.SemaphoreType.DMA,
        pltpu.SemaphoreType.DMA((2,)),
        pltpu.SemaphoreType.DMA((2,)),
    ],
)(_sc_body)


BLK = 2000


def _fin_body(s_ref, c_ref, o_ref):
    s = s_ref[0] + s_ref[1]
    c = c_ref[0] + c_ref[1]
    o_ref[...] = s / jnp.maximum(c, 1.0)


_finalize = pl.pallas_call(
    _fin_body,
    grid=(N // BLK,),
    in_specs=[
        pl.BlockSpec((NC, BLK, D), lambda i: (0, i, 0)),
        pl.BlockSpec((NC, BLK, 1), lambda i: (0, i, 0)),
    ],
    out_specs=pl.BlockSpec((BLK, D), lambda i: (i, 0)),
    out_shape=jax.ShapeDtypeStruct((N, D), jnp.float32),
)


def kernel(node_ids, edge_index, emb_weight):
    ei_p = edge_index.reshape(2, NGROUPS, G)
    sums, cnts = _sc_accumulate(ei_p, node_ids, emb_weight)
    return _finalize(sums, cnts.reshape(NC, ACC_ROWS, 1))
